# baseline (speedup 1.0000x reference)
import functools

import jax
import jax.numpy as jnp
from jax import lax
from jax.experimental import pallas as pl
from jax.experimental.pallas import tpu as pltpu
from jax.experimental.pallas import tpu_sc as plsc

N = 10000
E = 320000
DF = 128
DH = 64
DP = 16

NCORE = 2
NSUB = 16
NT = NCORE * NSUB

E_HALF = 163840
E_PAD = 2 * E_HALF

NPAD = 10112
NZ16 = NPAD // 16
RPT = NPAD // NSUB

CPT = 5
NCS = NT * CPT
CE = 2048
NCKV = E_HALF // CE
NG = CE // 16
GU = 4

CHUNK = 128
NCHUNK = E_HALF // (NSUB * CHUNK)
PCHUNK = NCHUNK // 2
XB = 4
PB = 2

BLK = 1024


def _scv_body(cols_hbm, idxi_hbm, idxj_hbm, out_hbm,
              cols_v, iis, jjs, acc_v, isems, jsems):
    c = lax.axis_index("c")
    s = lax.axis_index("s")
    w = c * NSUB + s
    myrows = pl.ds(w * CPT, CPT)

    pltpu.sync_copy(cols_hbm.at[myrows], cols_v)

    def zbody(k, carry):
        z = jnp.zeros((16,), jnp.float32)
        for cl in range(CPT):
            acc_v[cl, pl.ds(k * 16, 16)] = z
        return carry

    lax.fori_loop(0, NZ16, zbody, 0)

    def issue(ck, b):
        sl = pl.ds(ck * CE, CE)
        pltpu.async_copy(idxi_hbm.at[sl], iis[b], isems[b])
        pltpu.async_copy(idxj_hbm.at[sl], jjs[b], jsems[b])

    def wait(b):
        sl = pl.ds(0, CE)
        pltpu.make_async_copy(idxi_hbm.at[sl], iis[b], isems[b]).wait()
        pltpu.make_async_copy(idxj_hbm.at[sl], jjs[b], jsems[b]).wait()

    for b in range(2):
        issue(b, b)

    def chunk_body(p, carry):
        for b in range(2):
            ck = p * 2 + b
            wait(b)

            def gbody(g, carry2):
                iiu, jju, vals = [], [], []
                for u in range(GU):
                    e0 = (g * GU + u) * 16
                    iiu.append(iis[b][pl.ds(e0, 16)])
                    jju.append(jjs[b][pl.ds(e0, 16)])
                for u in range(GU):
                    for cl in range(CPT):
                        clv = jnp.full((16,), cl, jnp.int32)
                        vals.append(plsc.load_gather(cols_v, [clv, jju[u]]))
                for u in range(GU):
                    for cl in range(CPT):
                        clv = jnp.full((16,), cl, jnp.int32)
                        plsc.addupdate_scatter(acc_v, [clv, iiu[u]],
                                               vals[u * CPT + cl])
                return carry2

            lax.fori_loop(0, NG // GU, gbody, 0)

            @pl.when(ck + 2 < NCKV)
            def _():
                issue(ck + 2, b)

        return carry

    lax.fori_loop(0, NCKV // 2, chunk_body, 0)

    pltpu.sync_copy(acc_v, out_hbm.at[myrows])


@functools.cache
def _scv_segsum():
    return pl.kernel(
        _scv_body,
        out_type=jax.ShapeDtypeStruct((NCS, NPAD), jnp.float32),
        mesh=plsc.VectorSubcoreMesh(core_axis_name="c", subcore_axis_name="s"),
        compiler_params=pltpu.CompilerParams(use_tc_tiling_on_sc=False,
                                             needs_layout_passes=False),
        scratch_types=[
            pltpu.VMEM((CPT, N), jnp.float32),
            [pltpu.VMEM((CE,), jnp.int32) for _ in range(2)],
            [pltpu.VMEM((CE,), jnp.int32) for _ in range(2)],
            pltpu.VMEM((CPT, NPAD), jnp.float32),
            [pltpu.SemaphoreType.DMA for _ in range(2)],
            [pltpu.SemaphoreType.DMA for _ in range(2)],
        ],
    )


def _scs_body(xlo_hbm, xhi_hbm, pp_hbm, idxi_hbm, idxj_hbm,
              zx_hbm, zp_hbm, outx_hbm, outp_hbm,
              idxi_v, idxj_v, xbufs, pbufs, accx, accp, xsems, psems):
    c = lax.axis_index("c")
    s = lax.axis_index("s")

    rows = pl.ds(s * RPT, RPT)
    pltpu.sync_copy(zx_hbm.at[rows], accx.at[rows])
    pltpu.sync_copy(zp_hbm.at[rows], accp.at[rows])
    pltpu.sync_copy(idxi_hbm.at[s], idxi_v)
    pltpu.sync_copy(idxj_hbm.at[s], idxj_v)
    plsc.subcore_barrier()

    pbase = c * PCHUNK

    def x_issue(ch, b):
        @pl.when(c == 0)
        def _():
            pltpu.async_copy(xlo_hbm.at[idxj_v.at[ch]], xbufs[b], xsems[b])

        @pl.when(c != 0)
        def _():
            pltpu.async_copy(xhi_hbm.at[idxj_v.at[ch]], xbufs[b], xsems[b])

    def x_wait(b):
        pltpu.make_async_copy(xlo_hbm.at[idxj_v.at[0]], xbufs[b],
                              xsems[b]).wait()

    def p_issue(ch, b):
        pltpu.async_copy(pp_hbm.at[idxj_v.at[ch]], pbufs[b], psems[b])

    def p_wait(b):
        pltpu.make_async_copy(pp_hbm.at[idxj_v.at[0]], pbufs[b],
                              psems[b]).wait()

    for b in range(XB):
        x_issue(b, b)
    for b in range(PB):
        p_issue(pbase + b, b)

    def body(p, carry):
        for b in range(XB):
            ch = p * XB + b
            x_wait(b)
            pltpu.sync_copy(xbufs[b], accx.at[idxi_v.at[ch]], add=True)

            @pl.when(ch + XB < NCHUNK)
            def _():
                x_issue(ch + XB, b)

        for b in range(PB):
            q = p * PB + b
            p_wait(b)
            pltpu.sync_copy(pbufs[b], accp.at[idxi_v.at[pbase + q]], add=True)

            @pl.when(q + PB < PCHUNK)
            def _():
                p_issue(pbase + q + PB, b)

        return carry

    lax.fori_loop(0, NCHUNK // XB, body, 0)

    plsc.subcore_barrier()
    pltpu.sync_copy(accx.at[rows], outx_hbm.at[c, rows])
    pltpu.sync_copy(accp.at[rows], outp_hbm.at[c, rows])


@functools.cache
def _scs_segsum():
    return pl.kernel(
        _scs_body,
        out_type=(
            jax.ShapeDtypeStruct((NCORE, NPAD, DH), jnp.float32),
            jax.ShapeDtypeStruct((NCORE, NPAD, DP), jnp.float32),
        ),
        mesh=plsc.VectorSubcoreMesh(core_axis_name="c", subcore_axis_name="s"),
        compiler_params=pltpu.CompilerParams(use_tc_tiling_on_sc=False),
        scratch_types=[
            pltpu.VMEM((NCHUNK, CHUNK), jnp.int32),
            pltpu.VMEM((NCHUNK, CHUNK), jnp.int32),
            [pltpu.VMEM((CHUNK, DH), jnp.float32) for _ in range(XB)],
            [pltpu.VMEM((CHUNK, DP), jnp.float32) for _ in range(PB)],
            pltpu.VMEM_SHARED((NPAD, DH), jnp.float32),
            pltpu.VMEM_SHARED((NPAD, DP), jnp.float32),
            [pltpu.SemaphoreType.DMA for _ in range(XB)],
            [pltpu.SemaphoreType.DMA for _ in range(PB)],
        ],
    )


def _tc_dense_body(cols_ref, s_ref, sx_ref, sp_ref,
                   w1m_ref, b1m_ref, w1a_ref, b1a_ref,
                   wxi_ref, wdx_ref, wdp_ref, wctx_ref, b2m_ref,
                   w2ax_ref, w2aa_ref, b2a_ref, o_ref):
    f32 = jnp.float32

    def mm(a_ref, bT):
        return jnp.dot(a_ref[...], bT, preferred_element_type=f32)

    cols = cols_ref[...]
    xT = cols[0:DF]
    posT = cols[DF:DF + 3]
    st = s_ref[...]
    sxs = jnp.concatenate([sx_ref[0].T, sx_ref[1].T], axis=0)
    sps = (sp_ref[0] + sp_ref[1]).T
    sxT = st[0:DF] + sxs
    spT = st[DF:DF + 3] + sps[0:3]
    cntT = st[DF + 3:DF + 4] + sps[3:4]
    invT = 1.0 / jnp.maximum(cntT, 1.0)
    nzT = cntT > 0.0
    mean_xT = sxT * invT
    dposT = spT * invT - posT
    aggr1T = jnp.where(nzT, mm(w1m_ref, dposT) + b1m_ref[...], 0.0)
    ctxT = mm(w1a_ref, aggr1T) + b1a_ref[...]
    aggr2T = jnp.where(
        nzT,
        mm(wxi_ref, xT) + mm(wdx_ref, mean_xT) + mm(wdp_ref, dposT)
        + mm(wctx_ref, ctxT) + b2m_ref[...],
        0.0)
    outT = mm(w2ax_ref, xT) + mm(w2aa_ref, aggr2T) + b2a_ref[...]
    o_ref[...] = outT.T


def _tc_dense(cols, sums, sx, sp, *weights, interpret=False):
    def _full(a):
        return pl.BlockSpec(a.shape, lambda i: (0,) * a.ndim)

    return pl.pallas_call(
        _tc_dense_body,
        grid=(pl.cdiv(N, BLK),),
        in_specs=[
            pl.BlockSpec((NCS, BLK), lambda i: (0, i)),
            pl.BlockSpec((NCS, BLK), lambda i: (0, i)),
            pl.BlockSpec((NCORE, BLK, DH), lambda i: (0, i, 0)),
            pl.BlockSpec((NCORE, BLK, DP), lambda i: (0, i, 0)),
        ] + [_full(a) for a in weights],
        out_specs=pl.BlockSpec((BLK, DF), lambda i: (i, 0)),
        out_shape=jax.ShapeDtypeStruct((N, DF), jnp.float32),
        interpret=interpret,
    )(cols, sums, sx, sp, *weights)


def kernel(x, edge_index, pos, W1m, b1m, W1a, b1a, W2m, b2m, W2a, b2a):
    idx_i = edge_index[0].astype(jnp.int32)
    idx_j = edge_index[1].astype(jnp.int32)
    cols = jnp.concatenate(
        [x.T, pos.T, jnp.ones((1, N), jnp.float32),
         jnp.zeros((NCS - DF - 4, N), jnp.float32)], axis=0)
    pp = jnp.concatenate(
        [pos, jnp.ones((N, 1), jnp.float32),
         jnp.zeros((N, DP - 4), jnp.float32)], axis=1)
    padn = E_PAD - E
    idx_i = jnp.concatenate([idx_i, jnp.full((padn,), N, jnp.int32)])
    idx_j = jnp.concatenate([idx_j, jnp.zeros((padn,), jnp.int32)])
    vi, vj = idx_i[:E_HALF], idx_j[:E_HALF]
    si = idx_i[E_HALF:].reshape(NSUB, NCHUNK, CHUNK)
    sj = idx_j[E_HALF:].reshape(NSUB, NCHUNK, CHUNK)
    zx = jnp.zeros((NPAD, DH), jnp.float32)
    zp = jnp.zeros((NPAD, DP), jnp.float32)

    sums = _scv_segsum()(cols, vi, vj)
    sx, sp = _scs_segsum()(x[:, :DH], x[:, DH:], pp, si, sj, zx, zp)

    wxiT = (W2m[0:DF] - W2m[DF:2 * DF]).T
    return _tc_dense(
        cols, sums, sx, sp,
        W1m.T, b1m.reshape(-1, 1), W1a.T, b1a.reshape(-1, 1),
        wxiT, W2m[DF:2 * DF].T, W2m[2 * DF:2 * DF + 3].T,
        W2m[2 * DF + 3:].T, b2m.reshape(-1, 1),
        W2a[:DF].T, W2a[DF:].T, b2a.reshape(-1, 1))


# speedup vs baseline: 1.3504x; 1.3504x over previous
import functools

import jax
import jax.numpy as jnp
from jax import lax
from jax.experimental import pallas as pl
from jax.experimental.pallas import tpu as pltpu
from jax.experimental.pallas import tpu_sc as plsc

N = 10000
E = 320000
DF = 128

NCORE = 2
NSUB = 16
NT = NCORE * NSUB
CPT = 5
NCS = NT * CPT
CE = 2048
NCK = 160
E_PAD = NCK * CE
NG = CE // 16
GU = 8
NPAD = 10240
NZ16 = NPAD // 16

BLK = 1024


def _sc_body(cols_hbm, idxi_hbm, idxj_hbm, out_hbm,
             cols_v, iis, jjs, acc_v, isems, jsems):
    c = lax.axis_index("c")
    s = lax.axis_index("s")
    w = c * NSUB + s
    myrows = pl.ds(w * CPT, CPT)

    pltpu.sync_copy(cols_hbm.at[myrows], cols_v)

    def zbody(k, carry):
        z = jnp.zeros((16,), jnp.float32)
        for cl in range(CPT):
            acc_v[cl, pl.ds(k * 16, 16)] = z
        return carry

    lax.fori_loop(0, NZ16, zbody, 0)

    def issue(ck, b):
        sl = pl.ds(ck * CE, CE)
        pltpu.async_copy(idxi_hbm.at[sl], iis[b], isems[b])
        pltpu.async_copy(idxj_hbm.at[sl], jjs[b], jsems[b])

    def wait(b):
        sl = pl.ds(0, CE)
        pltpu.make_async_copy(idxi_hbm.at[sl], iis[b], isems[b]).wait()
        pltpu.make_async_copy(idxj_hbm.at[sl], jjs[b], jsems[b]).wait()

    for b in range(2):
        issue(b, b)

    def chunk_body(p, carry):
        for b in range(2):
            ck = p * 2 + b
            wait(b)

            def gbody(g, carry2):
                iiu, jju, vals = [], [], []
                for u in range(GU):
                    e0 = (g * GU + u) * 16
                    iiu.append(iis[b][pl.ds(e0, 16)])
                    jju.append(jjs[b][pl.ds(e0, 16)])
                for u in range(GU):
                    for cl in range(CPT):
                        clv = jnp.full((16,), cl, jnp.int32)
                        vals.append(plsc.load_gather(cols_v, [clv, jju[u]]))
                for u in range(GU):
                    for cl in range(CPT):
                        clv = jnp.full((16,), cl, jnp.int32)
                        plsc.addupdate_scatter(acc_v, [clv, iiu[u]],
                                               vals[u * CPT + cl])
                return carry2

            lax.fori_loop(0, NG // GU, gbody, 0)

            @pl.when(ck + 2 < NCK)
            def _():
                issue(ck + 2, b)

        return carry

    lax.fori_loop(0, NCK // 2, chunk_body, 0)

    pltpu.sync_copy(acc_v, out_hbm.at[myrows])


@functools.cache
def _sc_segsum():
    return pl.kernel(
        _sc_body,
        out_type=jax.ShapeDtypeStruct((NCS, NPAD), jnp.float32),
        mesh=plsc.VectorSubcoreMesh(core_axis_name="c", subcore_axis_name="s"),
        compiler_params=pltpu.CompilerParams(use_tc_tiling_on_sc=False, needs_layout_passes=False),
        scratch_types=[
            pltpu.VMEM((CPT, N), jnp.float32),
            [pltpu.VMEM((CE,), jnp.int32) for _ in range(2)],
            [pltpu.VMEM((CE,), jnp.int32) for _ in range(2)],
            pltpu.VMEM((CPT, NPAD), jnp.float32),
            [pltpu.SemaphoreType.DMA for _ in range(2)],
            [pltpu.SemaphoreType.DMA for _ in range(2)],
        ],
    )


def _tc_dense_body(cols_ref, s_ref,
                   w1m_ref, b1m_ref, w1a_ref, b1a_ref,
                   wxi_ref, wdx_ref, wdp_ref, wctx_ref, b2m_ref,
                   w2ax_ref, w2aa_ref, b2a_ref, o_ref):
    f32 = jnp.float32

    def mm(a_ref, bT):
        return jnp.dot(a_ref[...], bT, preferred_element_type=f32)

    cols = cols_ref[...]
    xT = cols[0:DF]
    posT = cols[DF:DF + 3]
    st = s_ref[...]
    sxT = st[0:DF]
    spT = st[DF:DF + 3]
    cntT = st[DF + 3:DF + 4]
    invT = 1.0 / jnp.maximum(cntT, 1.0)
    nzT = cntT > 0.0
    mean_xT = sxT * invT
    dposT = spT * invT - posT
    aggr1T = jnp.where(nzT, mm(w1m_ref, dposT) + b1m_ref[...], 0.0)
    ctxT = mm(w1a_ref, aggr1T) + b1a_ref[...]
    aggr2T = jnp.where(
        nzT,
        mm(wxi_ref, xT) + mm(wdx_ref, mean_xT) + mm(wdp_ref, dposT)
        + mm(wctx_ref, ctxT) + b2m_ref[...],
        0.0)
    outT = mm(w2ax_ref, xT) + mm(w2aa_ref, aggr2T) + b2a_ref[...]
    o_ref[...] = outT.T


def _tc_dense(cols, sums, *weights, interpret=False):
    def _full(a):
        return pl.BlockSpec(a.shape, lambda i: (0,) * a.ndim)

    return pl.pallas_call(
        _tc_dense_body,
        grid=(pl.cdiv(N, BLK),),
        in_specs=[
            pl.BlockSpec((NCS, BLK), lambda i: (0, i)),
            pl.BlockSpec((NCS, BLK), lambda i: (0, i)),
        ] + [_full(a) for a in weights],
        out_specs=pl.BlockSpec((BLK, DF), lambda i: (i, 0)),
        out_shape=jax.ShapeDtypeStruct((N, DF), jnp.float32),
        interpret=interpret,
    )(cols, sums, *weights)


def kernel(x, edge_index, pos, W1m, b1m, W1a, b1a, W2m, b2m, W2a, b2a):
    idx_i = edge_index[0].astype(jnp.int32)
    idx_j = edge_index[1].astype(jnp.int32)
    cols = jnp.concatenate(
        [x.T, pos.T, jnp.ones((1, N), jnp.float32),
         jnp.zeros((NCS - DF - 4, N), jnp.float32)], axis=0)
    padn = E_PAD - E
    idx_i = jnp.concatenate([idx_i, jnp.full((padn,), N, jnp.int32)])
    idx_j = jnp.concatenate([idx_j, jnp.zeros((padn,), jnp.int32)])

    sums = _sc_segsum()(cols, idx_i, idx_j)

    wxiT = (W2m[0:DF] - W2m[DF:2 * DF]).T
    return _tc_dense(
        cols, sums,
        W1m.T, b1m.reshape(-1, 1), W1a.T, b1a.reshape(-1, 1),
        wxiT, W2m[DF:2 * DF].T, W2m[2 * DF:2 * DF + 3].T,
        W2m[2 * DF + 3:].T, b2m.reshape(-1, 1),
        W2a[:DF].T, W2a[DF:].T, b2a.reshape(-1, 1))


# speedup vs baseline: 1.3505x; 1.0001x over previous
import functools

import jax
import jax.numpy as jnp
from jax import lax
from jax.experimental import pallas as pl
from jax.experimental.pallas import tpu as pltpu
from jax.experimental.pallas import tpu_sc as plsc

N = 10000
E = 320000
DF = 128

NCORE = 2
NSUB = 16
NT = NCORE * NSUB
CPT = 5
NCS = NT * CPT
CE = 2048
NCK = 160
E_PAD = NCK * CE
NG = CE // 16
GU = 4
NPAD = 10240
NZ16 = NPAD // 16

BLK = 1024


def _sc_body(cols_hbm, idxi_hbm, idxj_hbm, out_hbm,
             cols_v, iis, jjs, acc_v, isems, jsems):
    c = lax.axis_index("c")
    s = lax.axis_index("s")
    w = c * NSUB + s
    myrows = pl.ds(w * CPT, CPT)

    pltpu.sync_copy(cols_hbm.at[myrows], cols_v)

    def zbody(k, carry):
        z = jnp.zeros((16,), jnp.float32)
        for cl in range(CPT):
            acc_v[cl, pl.ds(k * 16, 16)] = z
        return carry

    lax.fori_loop(0, NZ16, zbody, 0)

    def issue(ck, b):
        sl = pl.ds(ck * CE, CE)
        pltpu.async_copy(idxi_hbm.at[sl], iis[b], isems[b])
        pltpu.async_copy(idxj_hbm.at[sl], jjs[b], jsems[b])

    def wait(b):
        sl = pl.ds(0, CE)
        pltpu.make_async_copy(idxi_hbm.at[sl], iis[b], isems[b]).wait()
        pltpu.make_async_copy(idxj_hbm.at[sl], jjs[b], jsems[b]).wait()

    for b in range(2):
        issue(b, b)

    def chunk_body(p, carry):
        for b in range(2):
            ck = p * 2 + b
            wait(b)

            def gbody(g, carry2):
                iiu, jju, vals = [], [], []
                for u in range(GU):
                    e0 = (g * GU + u) * 16
                    iiu.append(iis[b][pl.ds(e0, 16)])
                    jju.append(jjs[b][pl.ds(e0, 16)])
                for u in range(GU):
                    for cl in range(CPT):
                        clv = jnp.full((16,), cl, jnp.int32)
                        vals.append(plsc.load_gather(cols_v, [clv, jju[u]]))
                for u in range(GU):
                    for cl in range(CPT):
                        clv = jnp.full((16,), cl, jnp.int32)
                        plsc.addupdate_scatter(acc_v, [clv, iiu[u]],
                                               vals[u * CPT + cl])
                return carry2

            lax.fori_loop(0, NG // GU, gbody, 0)

            @pl.when(ck + 2 < NCK)
            def _():
                issue(ck + 2, b)

        return carry

    lax.fori_loop(0, NCK // 2, chunk_body, 0)

    pltpu.sync_copy(acc_v, out_hbm.at[myrows])


@functools.cache
def _sc_segsum():
    return pl.kernel(
        _sc_body,
        out_type=jax.ShapeDtypeStruct((NCS, NPAD), jnp.float32),
        mesh=plsc.VectorSubcoreMesh(core_axis_name="c", subcore_axis_name="s"),
        compiler_params=pltpu.CompilerParams(use_tc_tiling_on_sc=False, needs_layout_passes=False),
        scratch_types=[
            pltpu.VMEM((CPT, N), jnp.float32),
            [pltpu.VMEM((CE,), jnp.int32) for _ in range(2)],
            [pltpu.VMEM((CE,), jnp.int32) for _ in range(2)],
            pltpu.VMEM((CPT, NPAD), jnp.float32),
            [pltpu.SemaphoreType.DMA for _ in range(2)],
            [pltpu.SemaphoreType.DMA for _ in range(2)],
        ],
    )


def _tc_dense_body(cols_ref, s_ref,
                   w1m_ref, b1m_ref, w1a_ref, b1a_ref,
                   wxi_ref, wdx_ref, wdp_ref, wctx_ref, b2m_ref,
                   w2ax_ref, w2aa_ref, b2a_ref, o_ref):
    f32 = jnp.float32

    def mm(a_ref, bT):
        return jnp.dot(a_ref[...], bT, preferred_element_type=f32)

    cols = cols_ref[...]
    xT = cols[0:DF]
    posT = cols[DF:DF + 3]
    st = s_ref[...]
    sxT = st[0:DF]
    spT = st[DF:DF + 3]
    cntT = st[DF + 3:DF + 4]
    invT = 1.0 / jnp.maximum(cntT, 1.0)
    nzT = cntT > 0.0
    mean_xT = sxT * invT
    dposT = spT * invT - posT
    aggr1T = jnp.where(nzT, mm(w1m_ref, dposT) + b1m_ref[...], 0.0)
    ctxT = mm(w1a_ref, aggr1T) + b1a_ref[...]
    aggr2T = jnp.where(
        nzT,
        mm(wxi_ref, xT) + mm(wdx_ref, mean_xT) + mm(wdp_ref, dposT)
        + mm(wctx_ref, ctxT) + b2m_ref[...],
        0.0)
    outT = mm(w2ax_ref, xT) + mm(w2aa_ref, aggr2T) + b2a_ref[...]
    o_ref[...] = outT.T


def _tc_dense(cols, sums, *weights, interpret=False):
    def _full(a):
        return pl.BlockSpec(a.shape, lambda i: (0,) * a.ndim)

    return pl.pallas_call(
        _tc_dense_body,
        grid=(pl.cdiv(N, BLK),),
        in_specs=[
            pl.BlockSpec((NCS, BLK), lambda i: (0, i)),
            pl.BlockSpec((NCS, BLK), lambda i: (0, i)),
        ] + [_full(a) for a in weights],
        out_specs=pl.BlockSpec((BLK, DF), lambda i: (i, 0)),
        out_shape=jax.ShapeDtypeStruct((N, DF), jnp.float32),
        interpret=interpret,
    )(cols, sums, *weights)


def kernel(x, edge_index, pos, W1m, b1m, W1a, b1a, W2m, b2m, W2a, b2a):
    idx_i = edge_index[0].astype(jnp.int32)
    idx_j = edge_index[1].astype(jnp.int32)
    cols = jnp.concatenate(
        [x.T, pos.T, jnp.ones((1, N), jnp.float32),
         jnp.zeros((NCS - DF - 4, N), jnp.float32)], axis=0)
    padn = E_PAD - E
    idx_i = jnp.concatenate([idx_i, jnp.full((padn,), N, jnp.int32)])
    idx_j = jnp.concatenate([idx_j, jnp.zeros((padn,), jnp.int32)])

    sums = _sc_segsum()(cols, idx_i, idx_j)

    wxiT = (W2m[0:DF] - W2m[DF:2 * DF]).T
    return _tc_dense(
        cols, sums,
        W1m.T, b1m.reshape(-1, 1), W1a.T, b1a.reshape(-1, 1),
        wxiT, W2m[DF:2 * DF].T, W2m[2 * DF:2 * DF + 3].T,
        W2m[2 * DF + 3:].T, b2m.reshape(-1, 1),
        W2a[:DF].T, W2a[DF:].T, b2a.reshape(-1, 1))


# speedup vs baseline: 1.3640x; 1.0100x over previous
import functools

import jax
import jax.numpy as jnp
from jax import lax
from jax.experimental import pallas as pl
from jax.experimental.pallas import tpu as pltpu
from jax.experimental.pallas import tpu_sc as plsc

N = 10000
E = 320000
DF = 128

NCORE = 2
NSUB = 16
NT = NCORE * NSUB
CPT = 5
NCS = NT * CPT
CE = 2048
NCK = 160
E_PAD = NCK * CE
NG = CE // 16
GU = 4
NPAD = 10240
NZ16 = NPAD // 16

BLK = 1024


def _sc_body(cols_hbm, idxi_hbm, idxj_hbm, out_hbm,
             cols_v, iis, jjs, acc_v, isems, jsems):
    c = lax.axis_index("c")
    s = lax.axis_index("s")
    w = c * NSUB + s
    myrows = pl.ds(w * CPT, CPT)

    pltpu.sync_copy(cols_hbm.at[myrows], cols_v)

    def zbody(k, carry):
        z = jnp.zeros((16,), jnp.float32)
        for cl in range(CPT):
            acc_v[cl, pl.ds(k * 16, 16)] = z
        return carry

    lax.fori_loop(0, NZ16, zbody, 0)

    def issue(ck, b):
        sl = pl.ds(ck * CE, CE)
        pltpu.async_copy(idxi_hbm.at[sl], iis[b], isems[b])
        pltpu.async_copy(idxj_hbm.at[sl], jjs[b], jsems[b])

    def wait(b):
        sl = pl.ds(0, CE)
        pltpu.make_async_copy(idxi_hbm.at[sl], iis[b], isems[b]).wait()
        pltpu.make_async_copy(idxj_hbm.at[sl], jjs[b], jsems[b]).wait()

    for b in range(2):
        issue(b, b)

    def chunk_body(p, carry):
        for b in range(2):
            ck = p * 2 + b
            wait(b)

            def gbody(g, carry2):
                iiu, jju, vals = [], [], []
                for u in range(GU):
                    e0 = (g * GU + u) * 16
                    iiu.append(iis[b][pl.ds(e0, 16)])
                    jju.append(jjs[b][pl.ds(e0, 16)])
                for u in range(GU):
                    for cl in range(CPT):
                        clv = jnp.full((16,), cl, jnp.int32)
                        vals.append(plsc.load_gather(cols_v, [clv, jju[u]]))
                for u in range(GU):
                    for cl in range(CPT):
                        clv = jnp.full((16,), cl, jnp.int32)
                        plsc.addupdate_scatter(acc_v, [clv, iiu[u]],
                                               vals[u * CPT + cl])
                return carry2

            plsc.parallel_loop(0, NG // GU, 1, unroll=1)(
                lambda g: gbody(g, 0))

            @pl.when(ck + 2 < NCK)
            def _():
                issue(ck + 2, b)

        return carry

    lax.fori_loop(0, NCK // 2, chunk_body, 0)

    pltpu.sync_copy(acc_v, out_hbm.at[myrows])


@functools.cache
def _sc_segsum():
    return pl.kernel(
        _sc_body,
        out_type=jax.ShapeDtypeStruct((NCS, NPAD), jnp.float32),
        mesh=plsc.VectorSubcoreMesh(core_axis_name="c", subcore_axis_name="s"),
        compiler_params=pltpu.CompilerParams(use_tc_tiling_on_sc=False, needs_layout_passes=False),
        scratch_types=[
            pltpu.VMEM((CPT, N), jnp.float32),
            [pltpu.VMEM((CE,), jnp.int32) for _ in range(2)],
            [pltpu.VMEM((CE,), jnp.int32) for _ in range(2)],
            pltpu.VMEM((CPT, NPAD), jnp.float32),
            [pltpu.SemaphoreType.DMA for _ in range(2)],
            [pltpu.SemaphoreType.DMA for _ in range(2)],
        ],
    )


def _tc_dense_body(cols_ref, s_ref,
                   w1m_ref, b1m_ref, w1a_ref, b1a_ref,
                   wxi_ref, wdx_ref, wdp_ref, wctx_ref, b2m_ref,
                   w2ax_ref, w2aa_ref, b2a_ref, o_ref):
    f32 = jnp.float32

    def mm(a_ref, bT):
        return jnp.dot(a_ref[...], bT, preferred_element_type=f32)

    cols = cols_ref[...]
    xT = cols[0:DF]
    posT = cols[DF:DF + 3]
    st = s_ref[...]
    sxT = st[0:DF]
    spT = st[DF:DF + 3]
    cntT = st[DF + 3:DF + 4]
    invT = 1.0 / jnp.maximum(cntT, 1.0)
    nzT = cntT > 0.0
    mean_xT = sxT * invT
    dposT = spT * invT - posT
    aggr1T = jnp.where(nzT, mm(w1m_ref, dposT) + b1m_ref[...], 0.0)
    ctxT = mm(w1a_ref, aggr1T) + b1a_ref[...]
    aggr2T = jnp.where(
        nzT,
        mm(wxi_ref, xT) + mm(wdx_ref, mean_xT) + mm(wdp_ref, dposT)
        + mm(wctx_ref, ctxT) + b2m_ref[...],
        0.0)
    outT = mm(w2ax_ref, xT) + mm(w2aa_ref, aggr2T) + b2a_ref[...]
    o_ref[...] = outT.T


def _tc_dense(cols, sums, *weights, interpret=False):
    def _full(a):
        return pl.BlockSpec(a.shape, lambda i: (0,) * a.ndim)

    return pl.pallas_call(
        _tc_dense_body,
        grid=(pl.cdiv(N, BLK),),
        in_specs=[
            pl.BlockSpec((NCS, BLK), lambda i: (0, i)),
            pl.BlockSpec((NCS, BLK), lambda i: (0, i)),
        ] + [_full(a) for a in weights],
        out_specs=pl.BlockSpec((BLK, DF), lambda i: (i, 0)),
        out_shape=jax.ShapeDtypeStruct((N, DF), jnp.float32),
        interpret=interpret,
    )(cols, sums, *weights)


def kernel(x, edge_index, pos, W1m, b1m, W1a, b1a, W2m, b2m, W2a, b2a):
    idx_i = edge_index[0].astype(jnp.int32)
    idx_j = edge_index[1].astype(jnp.int32)
    cols = jnp.concatenate(
        [x.T, pos.T, jnp.ones((1, N), jnp.float32),
         jnp.zeros((NCS - DF - 4, N), jnp.float32)], axis=0)
    padn = E_PAD - E
    idx_i = jnp.concatenate([idx_i, jnp.full((padn,), N, jnp.int32)])
    idx_j = jnp.concatenate([idx_j, jnp.zeros((padn,), jnp.int32)])

    sums = _sc_segsum()(cols, idx_i, idx_j)

    wxiT = (W2m[0:DF] - W2m[DF:2 * DF]).T
    return _tc_dense(
        cols, sums,
        W1m.T, b1m.reshape(-1, 1), W1a.T, b1a.reshape(-1, 1),
        wxiT, W2m[DF:2 * DF].T, W2m[2 * DF:2 * DF + 3].T,
        W2m[2 * DF + 3:].T, b2m.reshape(-1, 1),
        W2a[:DF].T, W2a[DF:].T, b2a.reshape(-1, 1))
